# baseline (speedup 1.0000x reference)
import functools
import math

import jax
import jax.numpy as jnp
import numpy as np
from jax import lax
from jax.experimental import pallas as pl
from jax.experimental.pallas import tpu as pltpu
from jax.experimental.pallas import tpu_sc as plsc

_BATCH = 4096
_SEQ = 200
_D = 64
_B = _BATCH * _SEQ

_NC = 2
_NS = 16
_NW = _NC * _NS
_BPW = _B // _NW
_C = 512
_NCHUNK = _BPW // _C
_NBUF = 3
_TROWS = 32


def _idx_body(ts_ref, idx_ref):
    hours = ts_ref[...] // 3600
    cur = hours[:, _SEQ - 1:_SEQ]
    n = ((cur - hours) + 1).astype(jnp.float32)
    d = jnp.log(n) / np.float32(math.log(2))
    idx_ref[...] = jnp.ceil(d).astype(jnp.int32)


def _compute_idx(ts):
    blk = 256
    return pl.pallas_call(
        _idx_body,
        out_shape=jax.ShapeDtypeStruct((_BATCH, _SEQ), jnp.int32),
        grid=(_BATCH // blk,),
        in_specs=[pl.BlockSpec((blk, _SEQ), lambda i: (i, 0))],
        out_specs=pl.BlockSpec((blk, _SEQ), lambda i: (i, 0)),
    )(ts)


def _gather_body(idx_hbm, table_hbm, out_hbm, idx_v, table_v, rows_v, ssem):
    wid = lax.axis_index("s") * _NC + lax.axis_index("c")
    pltpu.sync_copy(table_hbm.at[pl.ds(0, _TROWS * _D // 128)], table_v)
    pltpu.sync_copy(idx_hbm.at[pl.ds(wid * (_BPW // 128), _BPW // 128)], idx_v)
    base2 = wid * (_BPW * _D // 128)
    crows = _C * _D // 128

    def chunk(i, carry):
        buf = lax.rem(i, _NBUF)

        def group(g, c2):
            r = i * (_C // 128) + lax.shift_right_logical(g, 3)
            c = lax.mul(lax.rem(g, 8), 16)
            vi = idx_v[r, pl.ds(c, 16)]
            prev = None
            for l0 in range(0, 16, 4):
                cur = []
                for l in range(l0, l0 + 4):
                    s = vi[l]
                    sr = lax.shift_right_logical(s, 1)
                    sc = lax.mul(lax.rem(s, 2), 64)
                    for t in range(4):
                        cur.append(
                            (l, t, table_v[sr, pl.ds(sc + t * 16, 16)])
                        )
                if prev is not None:
                    for l, t, v in prev:
                        rows_v[
                            buf, g * 8 + l // 2,
                            pl.ds((l % 2) * 64 + t * 16, 16),
                        ] = v
                prev = cur
            for l, t, v in prev:
                rows_v[
                    buf, g * 8 + l // 2,
                    pl.ds((l % 2) * 64 + t * 16, 16),
                ] = v
            return c2

        lax.fori_loop(0, _C // 16, group, 0)

        @pl.when(i >= _NBUF)
        def _():
            pltpu.make_async_copy(
                rows_v.at[buf], out_hbm.at[pl.ds(base2, crows)], ssem
            ).wait()

        pltpu.async_copy(
            rows_v.at[buf], out_hbm.at[pl.ds(base2 + i * crows, crows)], ssem
        )
        return carry

    lax.fori_loop(0, _NCHUNK, chunk, 0)
    for _ in range(_NBUF):
        pltpu.make_async_copy(
            rows_v.at[0], out_hbm.at[pl.ds(base2, crows)], ssem
        ).wait()


def _gather_sc(idx128, table128):
    mesh = plsc.VectorSubcoreMesh(
        core_axis_name="c", subcore_axis_name="s",
        num_cores=_NC, num_subcores=_NS,
    )
    f = functools.partial(
        pl.kernel,
        out_type=jax.ShapeDtypeStruct((_B * _D // 128, 128), jnp.float32),
        mesh=mesh,
        scratch_types=[
            pltpu.VMEM((_BPW // 128, 128), jnp.int32),
            pltpu.VMEM((_TROWS * _D // 128, 128), jnp.float32),
            pltpu.VMEM((_NBUF, _C * _D // 128, 128), jnp.float32),
            pltpu.SemaphoreType.DMA,
        ],
        compiler_params=pltpu.CompilerParams(use_tc_tiling_on_sc=True),
    )(_gather_body)
    return f(idx128, table128)


def kernel(timestamps, te_weight):
    ts = timestamps.astype(jnp.int32)
    idx = _compute_idx(ts)
    out = _gather_sc(idx.reshape(_B // 128, 128), te_weight.reshape(-1, 128))
    return out.reshape(_BATCH, _SEQ, _D)


# speedup vs baseline: 1.4375x; 1.4375x over previous
import functools
import math

import jax
import jax.numpy as jnp
import numpy as np
from jax import lax
from jax.experimental import pallas as pl
from jax.experimental.pallas import tpu as pltpu
from jax.experimental.pallas import tpu_sc as plsc

_BATCH = 4096
_SEQ = 200
_D = 64
_B = _BATCH * _SEQ

_NC = 2
_NS = 16
_NW = _NC * _NS
_BROWS = _BATCH // _NW
_BPW = _B // _NW
_CB = 2
_CROWS = _CB * _SEQ
_NCHUNK = _BROWS // _CB
_NBUF = 2
_TROWS = 32


def _idx_body(ts_ref, idx_ref):
    hours = ts_ref[...] // 3600
    cur = hours[:, _SEQ - 1:_SEQ]
    n = ((cur - hours) + 1).astype(jnp.float32)
    d = jnp.log(n) / np.float32(math.log(2))
    idx_ref[...] = jnp.ceil(d).astype(jnp.int32)


def _compute_idx(ts):
    blk = 256
    return pl.pallas_call(
        _idx_body,
        out_shape=jax.ShapeDtypeStruct((_BATCH, _SEQ), jnp.int32),
        grid=(_BATCH // blk,),
        in_specs=[pl.BlockSpec((blk, _SEQ), lambda i: (i, 0))],
        out_specs=pl.BlockSpec((blk, _SEQ), lambda i: (i, 0)),
    )(ts)


def _gather_body(idx_hbm, table_hbm, out_hbm, idx_v, table_v, rows_v, ssem):
    wid = lax.axis_index("s") * _NC + lax.axis_index("c")
    pltpu.sync_copy(table_hbm.at[pl.ds(0, _TROWS * _D // 128)], table_v)
    pltpu.sync_copy(idx_hbm.at[pl.ds(wid * (_BPW // 128), _BPW // 128)], idx_v)
    bbase = wid * _BROWS

    def chunk(c, carry):
        buf = lax.rem(c, _NBUF)

        def group(g, c2):
            u = c * (_CROWS // 16) + g
            r = lax.shift_right_logical(u, 3)
            col = lax.mul(lax.rem(u, 8), 16)
            vi = idx_v[r, pl.ds(col, 16)]
            fr0 = g * 16

            def dest(l):
                fr = fr0 + l
                q = jnp.where(fr >= _SEQ, 1, 0)
                srow = fr - q * _SEQ
                return q, srow

            prev = None
            for l0 in range(0, 16, 4):
                cur = []
                for l in range(l0, l0 + 4):
                    s = vi[l]
                    sr = lax.shift_right_logical(s, 1)
                    sc = lax.mul(lax.rem(s, 2), 64)
                    for t in range(4):
                        cur.append(
                            (l, t, table_v[sr, pl.ds(sc + t * 16, 16)])
                        )
                if prev is not None:
                    for l, t, v in prev:
                        q, srow = dest(l)
                        rows_v[buf, q, srow, pl.ds(t * 16, 16)] = v
                prev = cur
            for l, t, v in prev:
                q, srow = dest(l)
                rows_v[buf, q, srow, pl.ds(t * 16, 16)] = v
            return c2

        lax.fori_loop(0, _CROWS // 16, group, 0)

        @pl.when(c >= _NBUF)
        def _():
            pltpu.make_async_copy(
                rows_v.at[buf], out_hbm.at[pl.ds(bbase, _CB)], ssem
            ).wait()

        pltpu.async_copy(
            rows_v.at[buf], out_hbm.at[pl.ds(bbase + c * _CB, _CB)], ssem
        )
        return carry

    lax.fori_loop(0, _NCHUNK, chunk, 0)
    for _ in range(_NBUF):
        pltpu.make_async_copy(
            rows_v.at[0], out_hbm.at[pl.ds(bbase, _CB)], ssem
        ).wait()


def _gather_sc(idx128, table128):
    mesh = plsc.VectorSubcoreMesh(
        core_axis_name="c", subcore_axis_name="s",
        num_cores=_NC, num_subcores=_NS,
    )
    f = functools.partial(
        pl.kernel,
        out_type=jax.ShapeDtypeStruct((_BATCH, _SEQ, _D), jnp.float32),
        mesh=mesh,
        scratch_types=[
            pltpu.VMEM((_BPW // 128, 128), jnp.int32),
            pltpu.VMEM((_TROWS * _D // 128, 128), jnp.float32),
            pltpu.VMEM((_NBUF, _CB, _SEQ, _D), jnp.float32),
            pltpu.SemaphoreType.DMA,
        ],
        compiler_params=pltpu.CompilerParams(use_tc_tiling_on_sc=True),
    )(_gather_body)
    return f(idx128, table128)


def kernel(timestamps, te_weight):
    ts = timestamps.astype(jnp.int32)
    idx = _compute_idx(ts)
    return _gather_sc(idx.reshape(_B // 128, 128), te_weight.reshape(-1, 128))
